# baseline (speedup 1.0000x reference)
import jax
import jax.numpy as jnp
from jax.experimental import pallas as pl
from jax.experimental.pallas import tpu as pltpu

_NUM_SEGMENTS = 16
_BLK = 512


def _body(bounds_ref, seg_ref, x_ref, w_ref, b_ref, out_ref):
    i = pl.program_id(0)

    @pl.when(i == 0)
    def _init():
        out_ref[...] = jnp.zeros_like(out_ref)

    h = jnp.dot(
        x_ref[...].astype(jnp.bfloat16),
        w_ref[...].astype(jnp.bfloat16),
        preferred_element_type=jnp.float32,
    )
    h = jnp.maximum(h + b_ref[...], 0.0)
    seg = seg_ref[...]

    lo = bounds_ref[2 * i]
    hi = bounds_ref[2 * i + 1]

    def seg_step(s, carry):
        m = jnp.max(jnp.where(seg == s, h, 0.0), axis=0, keepdims=True)
        out_ref[pl.ds(s, 1), :] = jnp.maximum(out_ref[pl.ds(s, 1), :], m)
        return carry

    jax.lax.fori_loop(lo, hi + 1, seg_step, 0)


def kernel(x, W, b, batch):
    n, d_in = x.shape
    d_out = W.shape[1]
    g = n // _BLK
    seg = batch.astype(jnp.int32)
    bounds = jnp.stack([seg[::_BLK], seg[_BLK - 1 :: _BLK]], axis=1).reshape(-1)
    seg2d = seg.reshape(n, 1)

    grid_spec = pltpu.PrefetchScalarGridSpec(
        num_scalar_prefetch=1,
        grid=(g,),
        in_specs=[
            pl.BlockSpec((_BLK, 1), lambda i, *_: (i, 0)),
            pl.BlockSpec((_BLK, d_in), lambda i, *_: (i, 0)),
            pl.BlockSpec((d_in, d_out), lambda i, *_: (0, 0)),
            pl.BlockSpec((1, d_out), lambda i, *_: (0, 0)),
        ],
        out_specs=pl.BlockSpec((_NUM_SEGMENTS, d_out), lambda i, *_: (0, 0)),
    )
    out = pl.pallas_call(
        _body,
        grid_spec=grid_spec,
        out_shape=jax.ShapeDtypeStruct((_NUM_SEGMENTS, d_out), jnp.float32),
        compiler_params=pltpu.CompilerParams(
            dimension_semantics=("arbitrary",),
        ),
    )(bounds, seg2d, x, W, b.reshape(1, d_out))

    new_batch = jnp.arange(_NUM_SEGMENTS, dtype=jnp.int64)
    return (out, new_batch)


# speedup vs baseline: 1.2991x; 1.2991x over previous
import jax
import jax.numpy as jnp
from jax.experimental import pallas as pl
from jax.experimental.pallas import tpu as pltpu

_NUM_SEGMENTS = 16
_BLK = 1024


def _body(bounds_ref, seg_ref, x_ref, w_ref, b_ref, out_ref):
    i = pl.program_id(0)

    @pl.when(i == 0)
    def _init():
        out_ref[...] = jnp.zeros_like(out_ref)

    h = jnp.dot(
        x_ref[...].astype(jnp.bfloat16),
        w_ref[...].astype(jnp.bfloat16),
        preferred_element_type=jnp.float32,
    )
    h = jnp.maximum(h + b_ref[...], 0.0)
    seg = seg_ref[...]

    lo = bounds_ref[2 * i]
    hi = bounds_ref[2 * i + 1]

    def seg_step(s, carry):
        m = jnp.max(jnp.where(seg == s, h, 0.0), axis=0, keepdims=True)
        out_ref[pl.ds(s, 1), :] = jnp.maximum(out_ref[pl.ds(s, 1), :], m)
        return carry

    jax.lax.fori_loop(lo, hi + 1, seg_step, 0)


def kernel(x, W, b, batch):
    n, d_in = x.shape
    d_out = W.shape[1]
    g = n // _BLK
    seg = batch.astype(jnp.int32)
    bounds = jnp.stack([seg[::_BLK], seg[_BLK - 1 :: _BLK]], axis=1).reshape(-1)
    seg2d = seg.reshape(n, 1)

    grid_spec = pltpu.PrefetchScalarGridSpec(
        num_scalar_prefetch=1,
        grid=(g,),
        in_specs=[
            pl.BlockSpec((_BLK, 1), lambda i, *_: (i, 0)),
            pl.BlockSpec((_BLK, d_in), lambda i, *_: (i, 0)),
            pl.BlockSpec((d_in, d_out), lambda i, *_: (0, 0)),
            pl.BlockSpec((1, d_out), lambda i, *_: (0, 0)),
        ],
        out_specs=pl.BlockSpec((_NUM_SEGMENTS, d_out), lambda i, *_: (0, 0)),
    )
    out = pl.pallas_call(
        _body,
        grid_spec=grid_spec,
        out_shape=jax.ShapeDtypeStruct((_NUM_SEGMENTS, d_out), jnp.float32),
        compiler_params=pltpu.CompilerParams(
            dimension_semantics=("arbitrary",),
        ),
    )(bounds, seg2d, x, W, b.reshape(1, d_out))

    new_batch = jnp.arange(_NUM_SEGMENTS, dtype=jnp.int64)
    return (out, new_batch)


# speedup vs baseline: 1.3023x; 1.0024x over previous
import jax
import jax.numpy as jnp
from jax.experimental import pallas as pl
from jax.experimental.pallas import tpu as pltpu

_NUM_SEGMENTS = 16
_BLK = 2048


def _body(bounds_ref, seg_ref, x_ref, w_ref, b_ref, out_ref):
    i = pl.program_id(0)

    @pl.when(i == 0)
    def _init():
        out_ref[...] = jnp.zeros_like(out_ref)

    h = jnp.dot(
        x_ref[...].astype(jnp.bfloat16),
        w_ref[...].astype(jnp.bfloat16),
        preferred_element_type=jnp.float32,
    )
    h = jnp.maximum(h + b_ref[...], 0.0)
    seg = seg_ref[...]

    lo = bounds_ref[2 * i]
    hi = bounds_ref[2 * i + 1]

    def seg_step(s, carry):
        m = jnp.max(jnp.where(seg == s, h, 0.0), axis=0, keepdims=True)
        out_ref[pl.ds(s, 1), :] = jnp.maximum(out_ref[pl.ds(s, 1), :], m)
        return carry

    jax.lax.fori_loop(lo, hi + 1, seg_step, 0)


def kernel(x, W, b, batch):
    n, d_in = x.shape
    d_out = W.shape[1]
    g = n // _BLK
    seg = batch.astype(jnp.int32)
    bounds = jnp.stack([seg[::_BLK], seg[_BLK - 1 :: _BLK]], axis=1).reshape(-1)
    seg2d = seg.reshape(n, 1)

    grid_spec = pltpu.PrefetchScalarGridSpec(
        num_scalar_prefetch=1,
        grid=(g,),
        in_specs=[
            pl.BlockSpec((_BLK, 1), lambda i, *_: (i, 0)),
            pl.BlockSpec((_BLK, d_in), lambda i, *_: (i, 0)),
            pl.BlockSpec((d_in, d_out), lambda i, *_: (0, 0)),
            pl.BlockSpec((1, d_out), lambda i, *_: (0, 0)),
        ],
        out_specs=pl.BlockSpec((_NUM_SEGMENTS, d_out), lambda i, *_: (0, 0)),
    )
    out = pl.pallas_call(
        _body,
        grid_spec=grid_spec,
        out_shape=jax.ShapeDtypeStruct((_NUM_SEGMENTS, d_out), jnp.float32),
        compiler_params=pltpu.CompilerParams(
            dimension_semantics=("arbitrary",),
        ),
    )(bounds, seg2d, x, W, b.reshape(1, d_out))

    new_batch = jnp.arange(_NUM_SEGMENTS, dtype=jnp.int64)
    return (out, new_batch)
